# baseline (speedup 1.0000x reference)
import jax
import jax.numpy as jnp
from jax.experimental import pallas as pl
from jax.experimental.pallas import tpu as pltpu

N = 10000
D = 128
BM = 80


def _fused_kernel(x_ref, wm_ref, ws_ref, a1_ref, a2_ref, o1_ref, o2_ref,
                  b1_ref, b2_ref):
    @pl.when(pl.program_id(0) == 0)
    def _compute_b():
        x = x_ref[...]
        miu = jnp.dot(x, wm_ref[...], preferred_element_type=jnp.float32)
        sig = jnp.dot(x, ws_ref[...], preferred_element_type=jnp.float32)
        miu = jnp.where(miu > 0, miu, jnp.exp(jnp.minimum(miu, 0.0)) - 1.0)
        sig = jnp.maximum(sig, 0.0)
        att = jnp.exp(-sig)
        b1_ref[...] = miu * att
        b2_ref[...] = sig * att * att

    o1_ref[...] = jnp.dot(a1_ref[...], b1_ref[...],
                          preferred_element_type=jnp.float32)
    o2_ref[...] = jnp.dot(a2_ref[...], b2_ref[...],
                          preferred_element_type=jnp.float32)


@jax.jit
def kernel(features, adj_norm1, adj_norm2, weight_miu, weight_sigma):
    out1, out2 = pl.pallas_call(
        _fused_kernel,
        grid=(N // BM,),
        in_specs=[
            pl.BlockSpec((N, D), lambda i: (0, 0)),
            pl.BlockSpec((D, D), lambda i: (0, 0)),
            pl.BlockSpec((D, D), lambda i: (0, 0)),
            pl.BlockSpec((BM, N), lambda i: (i, 0)),
            pl.BlockSpec((BM, N), lambda i: (i, 0)),
        ],
        out_specs=[
            pl.BlockSpec((BM, D), lambda i: (i, 0)),
            pl.BlockSpec((BM, D), lambda i: (i, 0)),
        ],
        out_shape=[
            jax.ShapeDtypeStruct((N, D), jnp.float32),
            jax.ShapeDtypeStruct((N, D), jnp.float32),
        ],
        scratch_shapes=[
            pltpu.VMEM((N, D), jnp.float32),
            pltpu.VMEM((N, D), jnp.float32),
        ],
        compiler_params=pltpu.CompilerParams(
            dimension_semantics=("arbitrary",),
        ),
    )(features, weight_miu, weight_sigma, adj_norm1, adj_norm2)

    return (out1, out2)


# speedup vs baseline: 1.0397x; 1.0397x over previous
import jax
import jax.numpy as jnp
from jax.experimental import pallas as pl
from jax.experimental.pallas import tpu as pltpu

N = 10000
D = 128
BM = 200
BF = 2000


def _transform_kernel(x_ref, wm_ref, ws_ref, b1_ref, b2_ref):
    x = x_ref[...]
    miu = jnp.dot(x, wm_ref[...], preferred_element_type=jnp.float32)
    sig = jnp.dot(x, ws_ref[...], preferred_element_type=jnp.float32)
    miu = jnp.where(miu > 0, miu, jnp.exp(jnp.minimum(miu, 0.0)) - 1.0)
    sig = jnp.maximum(sig, 0.0)
    att = jnp.exp(-sig)
    b1_ref[...] = miu * att
    b2_ref[...] = sig * att * att


def _spmm_kernel(a1_ref, a2_ref, b1_ref, b2_ref, o1_ref, o2_ref):
    o1_ref[...] = jnp.dot(a1_ref[...], b1_ref[...],
                          preferred_element_type=jnp.float32)
    o2_ref[...] = jnp.dot(a2_ref[...], b2_ref[...],
                          preferred_element_type=jnp.float32)


@jax.jit
def kernel(features, adj_norm1, adj_norm2, weight_miu, weight_sigma):
    b1, b2 = pl.pallas_call(
        _transform_kernel,
        grid=(N // BF,),
        in_specs=[
            pl.BlockSpec((BF, D), lambda i: (i, 0)),
            pl.BlockSpec((D, D), lambda i: (0, 0)),
            pl.BlockSpec((D, D), lambda i: (0, 0)),
        ],
        out_specs=[
            pl.BlockSpec((BF, D), lambda i: (i, 0)),
            pl.BlockSpec((BF, D), lambda i: (i, 0)),
        ],
        out_shape=[
            jax.ShapeDtypeStruct((N, D), jnp.float32),
            jax.ShapeDtypeStruct((N, D), jnp.float32),
        ],
    )(features, weight_miu, weight_sigma)

    out1, out2 = pl.pallas_call(
        _spmm_kernel,
        grid=(N // BM,),
        in_specs=[
            pl.BlockSpec((BM, N), lambda i: (i, 0)),
            pl.BlockSpec((BM, N), lambda i: (i, 0)),
            pl.BlockSpec((N, D), lambda i: (0, 0)),
            pl.BlockSpec((N, D), lambda i: (0, 0)),
        ],
        out_specs=[
            pl.BlockSpec((BM, D), lambda i: (i, 0)),
            pl.BlockSpec((BM, D), lambda i: (i, 0)),
        ],
        out_shape=[
            jax.ShapeDtypeStruct((N, D), jnp.float32),
            jax.ShapeDtypeStruct((N, D), jnp.float32),
        ],
        compiler_params=pltpu.CompilerParams(
            dimension_semantics=("parallel",),
        ),
    )(adj_norm1, adj_norm2, b1, b2)

    return (out1, out2)


# speedup vs baseline: 1.0729x; 1.0320x over previous
import jax
import jax.numpy as jnp
from jax.experimental import pallas as pl
from jax.experimental.pallas import tpu as pltpu

N = 10000
D = 128
BM = 200


def _fused_kernel(x_ref, wm_ref, ws_ref, a1_ref, a2_ref, o1_ref, o2_ref,
                  b1_ref, b2_ref):
    @pl.when(pl.program_id(0) == 0)
    def _compute_b():
        x = x_ref[...]
        miu = jnp.dot(x, wm_ref[...], preferred_element_type=jnp.float32)
        sig = jnp.dot(x, ws_ref[...], preferred_element_type=jnp.float32)
        miu = jnp.where(miu > 0, miu, jnp.exp(jnp.minimum(miu, 0.0)) - 1.0)
        sig = jnp.maximum(sig, 0.0)
        att = jnp.exp(-sig)
        b1_ref[...] = miu * att
        b2_ref[...] = sig * att * att

    o1_ref[...] = jnp.dot(a1_ref[...], b1_ref[...],
                          preferred_element_type=jnp.float32)
    o2_ref[...] = jnp.dot(a2_ref[...], b2_ref[...],
                          preferred_element_type=jnp.float32)


@jax.jit
def kernel(features, adj_norm1, adj_norm2, weight_miu, weight_sigma):
    out1, out2 = pl.pallas_call(
        _fused_kernel,
        grid=(N // BM,),
        in_specs=[
            pl.BlockSpec((N, D), lambda i: (0, 0)),
            pl.BlockSpec((D, D), lambda i: (0, 0)),
            pl.BlockSpec((D, D), lambda i: (0, 0)),
            pl.BlockSpec((BM, N), lambda i: (i, 0)),
            pl.BlockSpec((BM, N), lambda i: (i, 0)),
        ],
        out_specs=[
            pl.BlockSpec((BM, D), lambda i: (i, 0)),
            pl.BlockSpec((BM, D), lambda i: (i, 0)),
        ],
        out_shape=[
            jax.ShapeDtypeStruct((N, D), jnp.float32),
            jax.ShapeDtypeStruct((N, D), jnp.float32),
        ],
        scratch_shapes=[
            pltpu.VMEM((N, D), jnp.float32),
            pltpu.VMEM((N, D), jnp.float32),
        ],
        compiler_params=pltpu.CompilerParams(
            dimension_semantics=("arbitrary",),
        ),
    )(features, weight_miu, weight_sigma, adj_norm1, adj_norm2)

    return (out1, out2)
